# baseline (speedup 1.0000x reference)
import functools

import jax
import jax.numpy as jnp
from jax import lax
from jax.experimental import pallas as pl
from jax.experimental.pallas import tpu as pltpu
from jax.experimental.pallas import tpu_sc as plsc

N_PAD = 10240
NC, NS = 2, 16
NW = NC * NS
CHUNK = 128
E_PAD = 327680
EPT = E_PAD // NW
NCHUNK = EPT // CHUNK
STRIPE = N_PAD // NS
ROWBLK = 1024
NBLK = N_PAD // ROWBLK
PMASK = 16383

_SC_PARAMS = pltpu.CompilerParams(needs_layout_passes=False)
_SC_MESH = dict(core_axis_name="c", subcore_axis_name="s")


DEGSZ = 2 * N_PAD
DSTRIPE = DEGSZ // NS


def _deg_body(pk_hbm, out_hbm, idx_v, deg_v, acc_v, tmp_v, parts_sh):
    cid = lax.axis_index("c")
    sid = lax.axis_index("s")
    wid = cid * NS + sid
    zero16 = jnp.zeros((16,), jnp.float32)

    def z16(i, carry):
        deg_v[pl.ds(i * 16, 16)] = zero16
        return carry

    lax.fori_loop(0, DEGSZ // 16, z16, 0)

    ones = jnp.full((16,), 1.0, jnp.float32)
    pltpu.sync_copy(pk_hbm.at[wid], idx_v)

    def acc(i, carry):
        v = idx_v[pl.ds(i * 16, 16)]
        plsc.addupdate_scatter(deg_v, [v & PMASK], ones)
        plsc.addupdate_scatter(deg_v, [(v >> 14) + N_PAD], ones)
        return carry

    lax.fori_loop(0, EPT // 16, acc, 0)

    pltpu.sync_copy(deg_v, parts_sh.at[sid])
    plsc.subcore_barrier()

    s0 = sid * DSTRIPE
    pltpu.sync_copy(parts_sh.at[0, pl.ds(s0, DSTRIPE)], acc_v)
    for p in range(1, NS):
        pltpu.sync_copy(parts_sh.at[p, pl.ds(s0, DSTRIPE)], tmp_v)

        def red(i, carry):
            sl = pl.ds(i * 16, 16)
            acc_v[sl] = acc_v[sl] + tmp_v[sl]
            return carry

        lax.fori_loop(0, DSTRIPE // 16, red, 0)
    pltpu.sync_copy(acc_v, out_hbm.at[cid, pl.ds(s0, DSTRIPE)])


_deg_kernel = functools.partial(
    pl.kernel,
    out_type=jax.ShapeDtypeStruct((NC, DEGSZ), jnp.float32),
    mesh=plsc.VectorSubcoreMesh(**_SC_MESH),
    compiler_params=_SC_PARAMS,
    scratch_types=[
        pltpu.VMEM((EPT,), jnp.int32),
        pltpu.VMEM((DEGSZ,), jnp.float32),
        pltpu.VMEM((DSTRIPE,), jnp.float32),
        pltpu.VMEM((DSTRIPE,), jnp.float32),
        pltpu.VMEM_SHARED((NS, DEGSZ), jnp.float32),
    ],
)(_deg_body)


def _agg_body(h_hbm, pk_hbm, zero_hbm, out_hbm,
              pk_v, si0, si1, di0, di1, buf_v, agg_sh, sem0, sem1):
    cid = lax.axis_index("c")
    sid = lax.axis_index("s")
    wid = cid * NS + sid
    pltpu.sync_copy(pk_hbm.at[wid], pk_v)

    sidx = (si0, si1)
    didx = (di0, di1)

    def unpack(j, slot):
        for k in range(CHUNK // 16):
            sl = pl.ds(k * 16, 16)
            v = pk_v[j, sl]
            sidx[slot][sl] = v & PMASK
            didx[slot][sl] = v >> 14

    r0 = sid * STRIPE
    pltpu.sync_copy(zero_hbm.at[pl.ds(r0, STRIPE)], agg_sh.at[pl.ds(r0, STRIPE)])
    plsc.subcore_barrier()

    unpack(0, 0)
    pltpu.async_copy(h_hbm.at[si0], buf_v.at[0], sem0)

    def step(i, carry):
        j0 = 2 * i
        j1 = 2 * i + 1
        unpack(j1, 1)
        pltpu.async_copy(h_hbm.at[si1], buf_v.at[1], sem1)
        pltpu.make_async_copy(h_hbm.at[si0], buf_v.at[0], sem0).wait()
        pltpu.sync_copy(buf_v.at[0], agg_sh.at[di0], add=True)

        @pl.when(j1 + 1 < NCHUNK)
        def _():
            unpack(j1 + 1, 0)
            pltpu.async_copy(h_hbm.at[si0], buf_v.at[0], sem0)

        pltpu.make_async_copy(h_hbm.at[si1], buf_v.at[1], sem1).wait()
        pltpu.sync_copy(buf_v.at[1], agg_sh.at[di1], add=True)
        return carry

    lax.fori_loop(0, NCHUNK // 2, step, 0)
    plsc.subcore_barrier()
    pltpu.sync_copy(agg_sh.at[pl.ds(r0, STRIPE)],
                    out_hbm.at[cid, pl.ds(r0, STRIPE)])


_agg_kernel = functools.partial(
    pl.kernel,
    out_type=jax.ShapeDtypeStruct((NC, N_PAD, 128), jnp.float32),
    mesh=plsc.VectorSubcoreMesh(**_SC_MESH),
    compiler_params=_SC_PARAMS,
    scratch_types=[
        pltpu.VMEM((NCHUNK, CHUNK), jnp.int32),
        pltpu.VMEM((CHUNK,), jnp.int32),
        pltpu.VMEM((CHUNK,), jnp.int32),
        pltpu.VMEM((CHUNK,), jnp.int32),
        pltpu.VMEM((CHUNK,), jnp.int32),
        pltpu.VMEM((2, CHUNK, 128), jnp.float32),
        pltpu.VMEM_SHARED((N_PAD, 128), jnp.float32),
        pltpu.SemaphoreType.DMA,
        pltpu.SemaphoreType.DMA,
    ],
)(_agg_body)


def _norms_from(deg_a, deg_b):
    deg = deg_a + deg_b
    return jnp.where(deg > 0, lax.rsqrt(jnp.maximum(deg, 1.0)), 0.0)


def _tc1_body(x_ref, w_ref, ds0, ds1, di0, di1, h_ref, no_ref, ni_ref):
    no = _norms_from(ds0[...], ds1[...])
    ni = _norms_from(di0[...], di1[...])
    no_ref[...] = no
    ni_ref[...] = ni
    h = jnp.dot(x_ref[...], w_ref[...], preferred_element_type=jnp.float32)
    h_ref[...] = h * no


def _tc1(x_p, w1, ds0, ds1, di0, di1):
    vec = pl.BlockSpec((ROWBLK, 1), lambda i: (i, 0))
    return pl.pallas_call(
        _tc1_body,
        grid=(NBLK,),
        in_specs=[
            pl.BlockSpec((ROWBLK, 128), lambda i: (i, 0)),
            pl.BlockSpec((128, 128), lambda i: (0, 0)),
            vec, vec, vec, vec,
        ],
        out_specs=[pl.BlockSpec((ROWBLK, 128), lambda i: (i, 0)), vec, vec],
        out_shape=[
            jax.ShapeDtypeStruct((N_PAD, 128), jnp.float32),
            jax.ShapeDtypeStruct((N_PAD, 1), jnp.float32),
            jax.ShapeDtypeStruct((N_PAD, 1), jnp.float32),
        ],
    )(x_p, w1, ds0, ds1, di0, di1)


def _tc2_body(p_ref, ni, no, b1r, w2, h2_ref):
    agg = p_ref[0] + p_ref[1]
    nic = ni[...]
    noc = no[...]
    t = jnp.maximum(agg * nic + b1r[...], 0.0)
    h2 = jnp.dot(t * noc, w2[...], preferred_element_type=jnp.float32)
    h2_ref[...] = jnp.concatenate(
        [h2, jnp.zeros((ROWBLK, 64), jnp.float32)], axis=1)


def _tc2(p, ni, no, b1r, w2):
    vec = pl.BlockSpec((ROWBLK, 1), lambda i: (i, 0))
    return pl.pallas_call(
        _tc2_body,
        grid=(NBLK,),
        in_specs=[
            pl.BlockSpec((NC, ROWBLK, 128), lambda i: (0, i, 0)),
            vec, vec,
            pl.BlockSpec((1, 128), lambda i: (0, 0)),
            pl.BlockSpec((128, 64), lambda i: (0, 0)),
        ],
        out_specs=pl.BlockSpec((ROWBLK, 128), lambda i: (i, 0)),
        out_shape=jax.ShapeDtypeStruct((N_PAD, 128), jnp.float32),
    )(p, ni, no, b1r, w2)


def _tc3_body(p_ref, ni, b2r, out_ref):
    z = (p_ref[0] + p_ref[1])[:, :64]
    nic = ni[...]
    z = z * nic + b2r[...]
    z = jnp.maximum(z, 0.0)
    m = jnp.max(z, axis=1, keepdims=True)
    e = jnp.exp(z - m)
    s = jnp.sum(e, axis=1, keepdims=True)
    out_ref[...] = (z - m) - jnp.log(s)


def _tc3(p, ni, b2r):
    vec = pl.BlockSpec((ROWBLK, 1), lambda i: (i, 0))
    return pl.pallas_call(
        _tc3_body,
        grid=(NBLK,),
        in_specs=[
            pl.BlockSpec((NC, ROWBLK, 128), lambda i: (0, i, 0)),
            vec,
            pl.BlockSpec((1, 64), lambda i: (0, 0)),
        ],
        out_specs=pl.BlockSpec((ROWBLK, 64), lambda i: (i, 0)),
        out_shape=jax.ShapeDtypeStruct((N_PAD, 64), jnp.float32),
    )(p, ni, b2r)


def kernel(features, edge_index, W1, b1, W2, b2):
    n, d_in = features.shape
    e = edge_index.shape[1]
    src = edge_index[0].astype(jnp.int32)
    dst = edge_index[1].astype(jnp.int32)
    pad_e = E_PAD - e
    pad_fill = n + jnp.arange(pad_e, dtype=jnp.int32) % (N_PAD - n)
    src_p = jnp.concatenate([src, pad_fill])
    dst_p = jnp.concatenate([dst, pad_fill])
    packed = src_p | (dst_p << 14)
    pk_deg = packed.reshape(NW, EPT)
    pk_agg = packed.reshape(NW, NCHUNK, CHUNK)
    x_p = jnp.pad(features, ((0, N_PAD - n), (0, 0)))
    zeros128 = jnp.zeros((N_PAD, 128), jnp.float32)

    degp = _deg_kernel(pk_deg)
    ds0 = degp[0, :N_PAD].reshape(N_PAD, 1)
    ds1 = degp[1, :N_PAD].reshape(N_PAD, 1)
    di0 = degp[0, N_PAD:].reshape(N_PAD, 1)
    di1 = degp[1, N_PAD:].reshape(N_PAD, 1)

    h1, no, ni = _tc1(x_p, W1, ds0, ds1, di0, di1)
    p1 = _agg_kernel(h1, pk_agg, zeros128)
    h2 = _tc2(p1, ni, no, b1.reshape(1, 128), W2)
    p2 = _agg_kernel(h2, pk_agg, zeros128)
    out = _tc3(p2, ni, b2.reshape(1, 64))
    return out[:n]


# speedup vs baseline: 1.0076x; 1.0076x over previous
import functools

import jax
import jax.numpy as jnp
from jax import lax
from jax.experimental import pallas as pl
from jax.experimental.pallas import tpu as pltpu
from jax.experimental.pallas import tpu_sc as plsc

N_PAD = 10240
NC, NS = 2, 16
NW = NC * NS
CHUNK = 128
E_PAD = 327680
EPT = E_PAD // NW
NCHUNK = EPT // CHUNK
STRIPE = N_PAD // NS
ROWBLK = 1024
NBLK = N_PAD // ROWBLK
PMASK = 16383

_SC_PARAMS = pltpu.CompilerParams(needs_layout_passes=False)
_SC_MESH = dict(core_axis_name="c", subcore_axis_name="s")


DEGSZ = 2 * N_PAD
DSTRIPE = DEGSZ // NS


def _deg_body(pk_hbm, out_hbm, idx_v, deg_v, acc_v, tmp_v, parts_sh):
    cid = lax.axis_index("c")
    sid = lax.axis_index("s")
    wid = cid * NS + sid
    zero16 = jnp.zeros((16,), jnp.float32)

    def z16(i, carry):
        deg_v[pl.ds(i * 16, 16)] = zero16
        return carry

    lax.fori_loop(0, DEGSZ // 16, z16, 0)

    ones = jnp.full((16,), 1.0, jnp.float32)
    pltpu.sync_copy(pk_hbm.at[wid], idx_v)

    def acc(i, carry):
        v = idx_v[pl.ds(i * 16, 16)]
        plsc.addupdate_scatter(deg_v, [v & PMASK], ones)
        plsc.addupdate_scatter(deg_v, [(v >> 14) + N_PAD], ones)
        return carry

    lax.fori_loop(0, EPT // 16, acc, 0)

    pltpu.sync_copy(deg_v, parts_sh.at[sid])
    plsc.subcore_barrier()

    s0 = sid * DSTRIPE
    pltpu.sync_copy(parts_sh.at[0, pl.ds(s0, DSTRIPE)], acc_v)
    for p in range(1, NS):
        pltpu.sync_copy(parts_sh.at[p, pl.ds(s0, DSTRIPE)], tmp_v)

        def red(i, carry):
            sl = pl.ds(i * 16, 16)
            acc_v[sl] = acc_v[sl] + tmp_v[sl]
            return carry

        lax.fori_loop(0, DSTRIPE // 16, red, 0)
    pltpu.sync_copy(acc_v, out_hbm.at[cid, pl.ds(s0, DSTRIPE)])


_deg_kernel = functools.partial(
    pl.kernel,
    out_type=jax.ShapeDtypeStruct((NC, DEGSZ), jnp.float32),
    mesh=plsc.VectorSubcoreMesh(**_SC_MESH),
    compiler_params=_SC_PARAMS,
    scratch_types=[
        pltpu.VMEM((EPT,), jnp.int32),
        pltpu.VMEM((DEGSZ,), jnp.float32),
        pltpu.VMEM((DSTRIPE,), jnp.float32),
        pltpu.VMEM((DSTRIPE,), jnp.float32),
        pltpu.VMEM_SHARED((NS, DEGSZ), jnp.float32),
    ],
)(_deg_body)


def _agg_body(h_hbm, pk_hbm, zero_hbm, out_hbm,
              pk_v, si0, si1, di0, di1, buf_v, agg_sh, sem0, sem1):
    cid = lax.axis_index("c")
    sid = lax.axis_index("s")
    wid = cid * NS + sid
    pltpu.sync_copy(pk_hbm.at[wid], pk_v)

    sidx = (si0, si1)
    didx = (di0, di1)

    def unpack(j, slot):
        for k in range(CHUNK // 16):
            sl = pl.ds(k * 16, 16)
            v = pk_v[j, sl]
            sidx[slot][sl] = v & PMASK
            didx[slot][sl] = v >> 14

    r0 = sid * STRIPE
    pltpu.sync_copy(zero_hbm.at[pl.ds(r0, STRIPE)], agg_sh.at[pl.ds(r0, STRIPE)])
    plsc.subcore_barrier()

    unpack(0, 0)
    pltpu.async_copy(h_hbm.at[si0], buf_v.at[0], sem0)

    def step(i, carry):
        j0 = 2 * i
        j1 = 2 * i + 1
        unpack(j1, 1)
        pltpu.async_copy(h_hbm.at[si1], buf_v.at[1], sem1)
        pltpu.make_async_copy(h_hbm.at[si0], buf_v.at[0], sem0).wait()
        pltpu.sync_copy(buf_v.at[0], agg_sh.at[di0], add=True)

        @pl.when(j1 + 1 < NCHUNK)
        def _():
            unpack(j1 + 1, 0)
            pltpu.async_copy(h_hbm.at[si0], buf_v.at[0], sem0)

        pltpu.make_async_copy(h_hbm.at[si1], buf_v.at[1], sem1).wait()
        pltpu.sync_copy(buf_v.at[1], agg_sh.at[di1], add=True)
        return carry

    lax.fori_loop(0, NCHUNK // 2, step, 0)
    plsc.subcore_barrier()
    pltpu.sync_copy(agg_sh.at[pl.ds(r0, STRIPE)],
                    out_hbm.at[cid, pl.ds(r0, STRIPE)])


_agg_kernel = functools.partial(
    pl.kernel,
    out_type=jax.ShapeDtypeStruct((NC, N_PAD, 128), jnp.float32),
    mesh=plsc.VectorSubcoreMesh(**_SC_MESH),
    compiler_params=_SC_PARAMS,
    scratch_types=[
        pltpu.VMEM((NCHUNK, CHUNK), jnp.int32),
        pltpu.VMEM((CHUNK,), jnp.int32),
        pltpu.VMEM((CHUNK,), jnp.int32),
        pltpu.VMEM((CHUNK,), jnp.int32),
        pltpu.VMEM((CHUNK,), jnp.int32),
        pltpu.VMEM((2, CHUNK, 128), jnp.float32),
        pltpu.VMEM_SHARED((N_PAD, 128), jnp.float32),
        pltpu.SemaphoreType.DMA,
        pltpu.SemaphoreType.DMA,
    ],
)(_agg_body)


def _norms_from(deg_a, deg_b):
    deg = deg_a + deg_b
    return jnp.where(deg > 0, lax.rsqrt(jnp.maximum(deg, 1.0)), 0.0)


def _mm1_body(x_ref, w_ref, xw_ref):
    xw_ref[...] = jnp.dot(x_ref[...], w_ref[...],
                          preferred_element_type=jnp.float32)


def _mm1(x_p, w1):
    return pl.pallas_call(
        _mm1_body,
        grid=(NBLK,),
        in_specs=[
            pl.BlockSpec((ROWBLK, 128), lambda i: (i, 0)),
            pl.BlockSpec((128, 128), lambda i: (0, 0)),
        ],
        out_specs=pl.BlockSpec((ROWBLK, 128), lambda i: (i, 0)),
        out_shape=jax.ShapeDtypeStruct((N_PAD, 128), jnp.float32),
    )(x_p, w1)


def _tc1_body(xw_ref, ds_ref, di_ref, h_ref, no_ref, ni_ref):
    no = _norms_from(ds_ref[0], ds_ref[1])
    ni = _norms_from(di_ref[0], di_ref[1])
    no_ref[...] = no
    ni_ref[...] = ni
    h_ref[...] = xw_ref[...] * no


def _tc1(xw, ds, di):
    vec3 = pl.BlockSpec((NC, ROWBLK, 1), lambda i: (0, i, 0))
    vec = pl.BlockSpec((ROWBLK, 1), lambda i: (i, 0))
    return pl.pallas_call(
        _tc1_body,
        grid=(NBLK,),
        in_specs=[
            pl.BlockSpec((ROWBLK, 128), lambda i: (i, 0)),
            vec3, vec3,
        ],
        out_specs=[pl.BlockSpec((ROWBLK, 128), lambda i: (i, 0)), vec, vec],
        out_shape=[
            jax.ShapeDtypeStruct((N_PAD, 128), jnp.float32),
            jax.ShapeDtypeStruct((N_PAD, 1), jnp.float32),
            jax.ShapeDtypeStruct((N_PAD, 1), jnp.float32),
        ],
    )(xw, ds, di)


def _tc2_body(p_ref, ni, no, b1r, w2, h2_ref):
    agg = p_ref[0] + p_ref[1]
    nic = ni[...]
    noc = no[...]
    t = jnp.maximum(agg * nic + b1r[...], 0.0)
    h2 = jnp.dot(t * noc, w2[...], preferred_element_type=jnp.float32)
    h2_ref[...] = jnp.concatenate(
        [h2, jnp.zeros((ROWBLK, 64), jnp.float32)], axis=1)


def _tc2(p, ni, no, b1r, w2):
    vec = pl.BlockSpec((ROWBLK, 1), lambda i: (i, 0))
    return pl.pallas_call(
        _tc2_body,
        grid=(NBLK,),
        in_specs=[
            pl.BlockSpec((NC, ROWBLK, 128), lambda i: (0, i, 0)),
            vec, vec,
            pl.BlockSpec((1, 128), lambda i: (0, 0)),
            pl.BlockSpec((128, 64), lambda i: (0, 0)),
        ],
        out_specs=pl.BlockSpec((ROWBLK, 128), lambda i: (i, 0)),
        out_shape=jax.ShapeDtypeStruct((N_PAD, 128), jnp.float32),
    )(p, ni, no, b1r, w2)


def _tc3_body(p_ref, ni, b2r, out_ref):
    z = (p_ref[0] + p_ref[1])[:, :64]
    nic = ni[...]
    z = z * nic + b2r[...]
    z = jnp.maximum(z, 0.0)
    m = jnp.max(z, axis=1, keepdims=True)
    e = jnp.exp(z - m)
    s = jnp.sum(e, axis=1, keepdims=True)
    out_ref[...] = (z - m) - jnp.log(s)


def _tc3(p, ni, b2r):
    vec = pl.BlockSpec((ROWBLK, 1), lambda i: (i, 0))
    return pl.pallas_call(
        _tc3_body,
        grid=(NBLK,),
        in_specs=[
            pl.BlockSpec((NC, ROWBLK, 128), lambda i: (0, i, 0)),
            vec,
            pl.BlockSpec((1, 64), lambda i: (0, 0)),
        ],
        out_specs=pl.BlockSpec((ROWBLK, 64), lambda i: (i, 0)),
        out_shape=jax.ShapeDtypeStruct((N_PAD, 64), jnp.float32),
    )(p, ni, b2r)


def kernel(features, edge_index, W1, b1, W2, b2):
    n, d_in = features.shape
    e = edge_index.shape[1]
    src = edge_index[0].astype(jnp.int32)
    dst = edge_index[1].astype(jnp.int32)
    pad_e = E_PAD - e
    pad_fill = n + jnp.arange(pad_e, dtype=jnp.int32) % (N_PAD - n)
    src_p = jnp.concatenate([src, pad_fill])
    dst_p = jnp.concatenate([dst, pad_fill])
    packed = src_p | (dst_p << 14)
    pk_deg = packed.reshape(NW, EPT)
    pk_agg = packed.reshape(NW, NCHUNK, CHUNK)
    x_p = jnp.pad(features, ((0, N_PAD - n), (0, 0)))
    zeros128 = jnp.zeros((N_PAD, 128), jnp.float32)

    degp = _deg_kernel(pk_deg)
    xw = _mm1(x_p, W1)
    degc = degp.reshape(NC, 2, N_PAD, 1)
    ds = degc[:, 0]
    di = degc[:, 1]

    h1, no, ni = _tc1(xw, ds, di)
    p1 = _agg_kernel(h1, pk_agg, zeros128)
    h2 = _tc2(p1, ni, no, b1.reshape(1, 128), W2)
    p2 = _agg_kernel(h2, pk_agg, zeros128)
    out = _tc3(p2, ni, b2.reshape(1, 64))
    return out[:n]


# speedup vs baseline: 1.0195x; 1.0119x over previous
import functools

import jax
import jax.numpy as jnp
from jax import lax
from jax.experimental import pallas as pl
from jax.experimental.pallas import tpu as pltpu
from jax.experimental.pallas import tpu_sc as plsc

N_PAD = 10240
NC, NS = 2, 16
NW = NC * NS
CHUNK = 128
E_PAD = 327680
EPT = E_PAD // NW
NCHUNK = EPT // CHUNK
STRIPE = N_PAD // NS
ROWBLK = 1024
NBLK = N_PAD // ROWBLK
PMASK = 16383

_SC_PARAMS = pltpu.CompilerParams(needs_layout_passes=False)
_SC_MESH = dict(core_axis_name="c", subcore_axis_name="s")


DEGSZ = 2 * N_PAD
DSTRIPE = DEGSZ // NS


def _deg_body(pk_hbm, zd_hbm, out_hbm, idx_v, deg_v, acc_v, tmp_v, parts_sh):
    cid = lax.axis_index("c")
    sid = lax.axis_index("s")
    wid = cid * NS + sid
    pltpu.sync_copy(zd_hbm, deg_v)

    ones = jnp.full((16,), 1.0, jnp.float32)
    pltpu.sync_copy(pk_hbm.at[wid], idx_v)

    def acc(i, carry):
        v = idx_v[pl.ds(i * 16, 16)]
        plsc.addupdate_scatter(deg_v, [v & PMASK], ones)
        plsc.addupdate_scatter(deg_v, [(v >> 14) + N_PAD], ones)
        return carry

    lax.fori_loop(0, EPT // 16, acc, 0)

    pltpu.sync_copy(deg_v, parts_sh.at[sid])
    plsc.subcore_barrier()

    s0 = sid * DSTRIPE
    pltpu.sync_copy(parts_sh.at[:, pl.ds(s0, DSTRIPE)], tmp_v)

    def red(i, carry):
        sl = pl.ds(i * 16, 16)
        s = tmp_v[0, sl]
        for p in range(1, NS):
            s = s + tmp_v[p, sl]
        acc_v[sl] = s
        return carry

    lax.fori_loop(0, DSTRIPE // 16, red, 0)
    pltpu.sync_copy(acc_v, out_hbm.at[cid, pl.ds(s0, DSTRIPE)])


_deg_kernel = functools.partial(
    pl.kernel,
    out_type=jax.ShapeDtypeStruct((NC, DEGSZ), jnp.float32),
    mesh=plsc.VectorSubcoreMesh(**_SC_MESH),
    compiler_params=_SC_PARAMS,
    scratch_types=[
        pltpu.VMEM((EPT,), jnp.int32),
        pltpu.VMEM((DEGSZ,), jnp.float32),
        pltpu.VMEM((DSTRIPE,), jnp.float32),
        pltpu.VMEM((NS, DSTRIPE), jnp.float32),
        pltpu.VMEM_SHARED((NS, DEGSZ), jnp.float32),
    ],
)(_deg_body)


def _agg_body(h_hbm, pk_hbm, zero_hbm, out_hbm,
              pk_v, si0, si1, di0, di1, buf_v, agg_sh, sem0, sem1):
    cid = lax.axis_index("c")
    sid = lax.axis_index("s")
    wid = cid * NS + sid
    pltpu.sync_copy(pk_hbm.at[wid], pk_v)

    sidx = (si0, si1)
    didx = (di0, di1)

    def unpack(j, slot):
        for k in range(CHUNK // 16):
            sl = pl.ds(k * 16, 16)
            v = pk_v[j, sl]
            sidx[slot][sl] = v & PMASK
            didx[slot][sl] = v >> 14

    r0 = sid * STRIPE
    pltpu.sync_copy(zero_hbm.at[pl.ds(r0, STRIPE)], agg_sh.at[pl.ds(r0, STRIPE)])
    plsc.subcore_barrier()

    unpack(0, 0)
    pltpu.async_copy(h_hbm.at[si0], buf_v.at[0], sem0)

    def step(i, carry):
        j0 = 2 * i
        j1 = 2 * i + 1
        unpack(j1, 1)
        pltpu.async_copy(h_hbm.at[si1], buf_v.at[1], sem1)
        pltpu.make_async_copy(h_hbm.at[si0], buf_v.at[0], sem0).wait()
        pltpu.sync_copy(buf_v.at[0], agg_sh.at[di0], add=True)

        @pl.when(j1 + 1 < NCHUNK)
        def _():
            unpack(j1 + 1, 0)
            pltpu.async_copy(h_hbm.at[si0], buf_v.at[0], sem0)

        pltpu.make_async_copy(h_hbm.at[si1], buf_v.at[1], sem1).wait()
        pltpu.sync_copy(buf_v.at[1], agg_sh.at[di1], add=True)
        return carry

    lax.fori_loop(0, NCHUNK // 2, step, 0)
    plsc.subcore_barrier()
    pltpu.sync_copy(agg_sh.at[pl.ds(r0, STRIPE)],
                    out_hbm.at[cid, pl.ds(r0, STRIPE)])


_agg_kernel = functools.partial(
    pl.kernel,
    out_type=jax.ShapeDtypeStruct((NC, N_PAD, 128), jnp.float32),
    mesh=plsc.VectorSubcoreMesh(**_SC_MESH),
    compiler_params=_SC_PARAMS,
    scratch_types=[
        pltpu.VMEM((NCHUNK, CHUNK), jnp.int32),
        pltpu.VMEM((CHUNK,), jnp.int32),
        pltpu.VMEM((CHUNK,), jnp.int32),
        pltpu.VMEM((CHUNK,), jnp.int32),
        pltpu.VMEM((CHUNK,), jnp.int32),
        pltpu.VMEM((2, CHUNK, 128), jnp.float32),
        pltpu.VMEM_SHARED((N_PAD, 128), jnp.float32),
        pltpu.SemaphoreType.DMA,
        pltpu.SemaphoreType.DMA,
    ],
)(_agg_body)


def _norms_from(deg_a, deg_b):
    deg = deg_a + deg_b
    return jnp.where(deg > 0, lax.rsqrt(jnp.maximum(deg, 1.0)), 0.0)


def _mm1_body(x_ref, w_ref, xw_ref):
    xw_ref[...] = jnp.dot(x_ref[...], w_ref[...],
                          preferred_element_type=jnp.float32)


def _mm1(x_p, w1):
    return pl.pallas_call(
        _mm1_body,
        grid=(NBLK,),
        in_specs=[
            pl.BlockSpec((ROWBLK, 128), lambda i: (i, 0)),
            pl.BlockSpec((128, 128), lambda i: (0, 0)),
        ],
        out_specs=pl.BlockSpec((ROWBLK, 128), lambda i: (i, 0)),
        out_shape=jax.ShapeDtypeStruct((N_PAD, 128), jnp.float32),
    )(x_p, w1)


def _tc1_body(xw_ref, ds_ref, di_ref, h_ref, no_ref, ni_ref):
    no = _norms_from(ds_ref[0], ds_ref[1])
    ni = _norms_from(di_ref[0], di_ref[1])
    no_ref[...] = no
    ni_ref[...] = ni
    h_ref[...] = xw_ref[...] * no


def _tc1(xw, ds, di):
    vec3 = pl.BlockSpec((NC, ROWBLK, 1), lambda i: (0, i, 0))
    vec = pl.BlockSpec((ROWBLK, 1), lambda i: (i, 0))
    return pl.pallas_call(
        _tc1_body,
        grid=(NBLK,),
        in_specs=[
            pl.BlockSpec((ROWBLK, 128), lambda i: (i, 0)),
            vec3, vec3,
        ],
        out_specs=[pl.BlockSpec((ROWBLK, 128), lambda i: (i, 0)), vec, vec],
        out_shape=[
            jax.ShapeDtypeStruct((N_PAD, 128), jnp.float32),
            jax.ShapeDtypeStruct((N_PAD, 1), jnp.float32),
            jax.ShapeDtypeStruct((N_PAD, 1), jnp.float32),
        ],
    )(xw, ds, di)


def _tc2_body(p_ref, ni, no, b1r, w2, h2_ref):
    agg = p_ref[0] + p_ref[1]
    nic = ni[...]
    noc = no[...]
    t = jnp.maximum(agg * nic + b1r[...], 0.0)
    h2 = jnp.dot(t * noc, w2[...], preferred_element_type=jnp.float32)
    h2_ref[...] = jnp.concatenate(
        [h2, jnp.zeros((ROWBLK, 64), jnp.float32)], axis=1)


def _tc2(p, ni, no, b1r, w2):
    vec = pl.BlockSpec((ROWBLK, 1), lambda i: (i, 0))
    return pl.pallas_call(
        _tc2_body,
        grid=(NBLK,),
        in_specs=[
            pl.BlockSpec((NC, ROWBLK, 128), lambda i: (0, i, 0)),
            vec, vec,
            pl.BlockSpec((1, 128), lambda i: (0, 0)),
            pl.BlockSpec((128, 64), lambda i: (0, 0)),
        ],
        out_specs=pl.BlockSpec((ROWBLK, 128), lambda i: (i, 0)),
        out_shape=jax.ShapeDtypeStruct((N_PAD, 128), jnp.float32),
    )(p, ni, no, b1r, w2)


def _tc3_body(p_ref, ni, b2r, out_ref):
    z = (p_ref[0] + p_ref[1])[:, :64]
    nic = ni[...]
    z = z * nic + b2r[...]
    z = jnp.maximum(z, 0.0)
    m = jnp.max(z, axis=1, keepdims=True)
    e = jnp.exp(z - m)
    s = jnp.sum(e, axis=1, keepdims=True)
    out_ref[...] = (z - m) - jnp.log(s)


def _tc3(p, ni, b2r):
    vec = pl.BlockSpec((ROWBLK, 1), lambda i: (i, 0))
    return pl.pallas_call(
        _tc3_body,
        grid=(NBLK,),
        in_specs=[
            pl.BlockSpec((NC, ROWBLK, 128), lambda i: (0, i, 0)),
            vec,
            pl.BlockSpec((1, 64), lambda i: (0, 0)),
        ],
        out_specs=pl.BlockSpec((ROWBLK, 64), lambda i: (i, 0)),
        out_shape=jax.ShapeDtypeStruct((N_PAD, 64), jnp.float32),
    )(p, ni, b2r)


def kernel(features, edge_index, W1, b1, W2, b2):
    n, d_in = features.shape
    e = edge_index.shape[1]
    src = edge_index[0].astype(jnp.int32)
    dst = edge_index[1].astype(jnp.int32)
    pad_e = E_PAD - e
    pad_fill = n + jnp.arange(pad_e, dtype=jnp.int32) % (N_PAD - n)
    src_p = jnp.concatenate([src, pad_fill])
    dst_p = jnp.concatenate([dst, pad_fill])
    packed = src_p | (dst_p << 14)
    pk_deg = packed.reshape(NW, EPT)
    pk_agg = packed.reshape(NW, NCHUNK, CHUNK)
    x_p = jnp.pad(features, ((0, N_PAD - n), (0, 0)))
    zeros128 = jnp.zeros((N_PAD, 128), jnp.float32)

    zd = jnp.zeros((DEGSZ,), jnp.float32)
    degp = _deg_kernel(pk_deg, zd)
    xw = _mm1(x_p, W1)
    degc = degp.reshape(NC, 2, N_PAD, 1)
    ds = degc[:, 0]
    di = degc[:, 1]

    h1, no, ni = _tc1(xw, ds, di)
    p1 = _agg_kernel(h1, pk_agg, zeros128)
    h2 = _tc2(p1, ni, no, b1.reshape(1, 128), W2)
    p2 = _agg_kernel(h2, pk_agg, zeros128)
    out = _tc3(p2, ni, b2.reshape(1, 64))
    return out[:n]


# speedup vs baseline: 1.0975x; 1.0765x over previous
import functools

import jax
import jax.numpy as jnp
from jax import lax
from jax.experimental import pallas as pl
from jax.experimental.pallas import tpu as pltpu
from jax.experimental.pallas import tpu_sc as plsc

N_PAD = 10240
NC, NS = 2, 16
NW = NC * NS
CHUNK = 128
E_PAD = 327680
EPT = E_PAD // NW
NCHUNK = EPT // CHUNK
STRIPE = N_PAD // NS
ROWBLK = 1024
NBLK = N_PAD // ROWBLK
PMASK = 16383

_SC_PARAMS = pltpu.CompilerParams(needs_layout_passes=False)
_SC_MESH = dict(core_axis_name="c", subcore_axis_name="s")


DEGSZ = 2 * N_PAD
DSTRIPE = DEGSZ // NS


def _deg_body(pk_hbm, zd_hbm, out_hbm, idx_v, deg_v, acc_v, tmp_v, parts_sh):
    cid = lax.axis_index("c")
    sid = lax.axis_index("s")
    wid = cid * NS + sid
    pltpu.sync_copy(zd_hbm, deg_v)

    ones = jnp.full((16,), 1.0, jnp.float32)
    pltpu.sync_copy(pk_hbm.at[wid], idx_v)

    def acc(i, carry):
        v = idx_v[pl.ds(i * 16, 16)]
        plsc.addupdate_scatter(deg_v, [v & PMASK], ones)
        plsc.addupdate_scatter(deg_v, [(v >> 14) + N_PAD], ones)
        return carry

    lax.fori_loop(0, EPT // 16, acc, 0)

    pltpu.sync_copy(deg_v, parts_sh.at[sid])
    plsc.subcore_barrier()

    s0 = sid * DSTRIPE
    pltpu.sync_copy(parts_sh.at[:, pl.ds(s0, DSTRIPE)], tmp_v)

    def red(i, carry):
        sl = pl.ds(i * 16, 16)
        s = tmp_v[0, sl]
        for p in range(1, NS):
            s = s + tmp_v[p, sl]
        acc_v[sl] = s
        return carry

    lax.fori_loop(0, DSTRIPE // 16, red, 0)
    pltpu.sync_copy(acc_v, out_hbm.at[cid, pl.ds(s0, DSTRIPE)])


_deg_kernel = functools.partial(
    pl.kernel,
    out_type=jax.ShapeDtypeStruct((NC, DEGSZ), jnp.float32),
    mesh=plsc.VectorSubcoreMesh(**_SC_MESH),
    compiler_params=_SC_PARAMS,
    scratch_types=[
        pltpu.VMEM((EPT,), jnp.int32),
        pltpu.VMEM((DEGSZ,), jnp.float32),
        pltpu.VMEM((DSTRIPE,), jnp.float32),
        pltpu.VMEM((NS, DSTRIPE), jnp.float32),
        pltpu.VMEM_SHARED((NS, DEGSZ), jnp.float32),
    ],
)(_deg_body)


def _agg_body(h_hbm, pk_hbm, zero_hbm, out_hbm,
              pk_v, si0, si1, di0, di1, buf_v, agg_sh, sem0, sem1):
    cid = lax.axis_index("c")
    sid = lax.axis_index("s")
    wid = cid * NS + sid
    pltpu.sync_copy(pk_hbm.at[wid], pk_v)

    sidx = (si0, si1)
    didx = (di0, di1)

    def unpack(j, slot):
        for k in range(CHUNK // 16):
            sl = pl.ds(k * 16, 16)
            v = pk_v[j, sl]
            sidx[slot][sl] = v & PMASK
            didx[slot][sl] = v >> 14

    r0 = sid * STRIPE
    pltpu.sync_copy(zero_hbm.at[pl.ds(r0, STRIPE)], agg_sh.at[pl.ds(r0, STRIPE)])
    plsc.subcore_barrier()

    unpack(0, 0)
    pltpu.async_copy(h_hbm.at[si0], buf_v.at[0], sem0)

    def step(i, carry):
        j0 = 2 * i
        j1 = 2 * i + 1
        unpack(j1, 1)
        pltpu.async_copy(h_hbm.at[si1], buf_v.at[1], sem1)
        pltpu.make_async_copy(h_hbm.at[si0], buf_v.at[0], sem0).wait()
        pltpu.sync_copy(buf_v.at[0], agg_sh.at[di0], add=True)

        @pl.when(j1 + 1 < NCHUNK)
        def _():
            unpack(j1 + 1, 0)
            pltpu.async_copy(h_hbm.at[si0], buf_v.at[0], sem0)

        pltpu.make_async_copy(h_hbm.at[si1], buf_v.at[1], sem1).wait()
        pltpu.sync_copy(buf_v.at[1], agg_sh.at[di1], add=True)
        return carry

    lax.fori_loop(0, NCHUNK // 2, step, 0)
    plsc.subcore_barrier()
    pltpu.sync_copy(agg_sh.at[pl.ds(r0, STRIPE)],
                    out_hbm.at[cid, pl.ds(r0, STRIPE)])


def _make_agg_kernel(dh, tc_tiling):
    return functools.partial(
        pl.kernel,
        out_type=jax.ShapeDtypeStruct((NC, N_PAD, dh), jnp.float32),
        mesh=plsc.VectorSubcoreMesh(**_SC_MESH),
        compiler_params=pltpu.CompilerParams(
            needs_layout_passes=False, use_tc_tiling_on_sc=tc_tiling),
        scratch_types=[
            pltpu.VMEM((NCHUNK, CHUNK), jnp.int32),
            pltpu.VMEM((CHUNK,), jnp.int32),
            pltpu.VMEM((CHUNK,), jnp.int32),
            pltpu.VMEM((CHUNK,), jnp.int32),
            pltpu.VMEM((CHUNK,), jnp.int32),
            pltpu.VMEM((2, CHUNK, dh), jnp.float32),
            pltpu.VMEM_SHARED((N_PAD, dh), jnp.float32),
            pltpu.SemaphoreType.DMA,
            pltpu.SemaphoreType.DMA,
        ],
    )(_agg_body)


_agg_kernel = _make_agg_kernel(128, None)
_agg_kernel64 = _make_agg_kernel(64, False)


def _norms_from(deg_a, deg_b):
    deg = deg_a + deg_b
    return jnp.where(deg > 0, lax.rsqrt(jnp.maximum(deg, 1.0)), 0.0)


def _mm1_body(x_ref, w_ref, xw_ref):
    xw_ref[...] = jnp.dot(x_ref[...], w_ref[...],
                          preferred_element_type=jnp.float32)


def _mm1(x_p, w1):
    return pl.pallas_call(
        _mm1_body,
        grid=(NBLK,),
        in_specs=[
            pl.BlockSpec((ROWBLK, 128), lambda i: (i, 0)),
            pl.BlockSpec((128, 128), lambda i: (0, 0)),
        ],
        out_specs=pl.BlockSpec((ROWBLK, 128), lambda i: (i, 0)),
        out_shape=jax.ShapeDtypeStruct((N_PAD, 128), jnp.float32),
    )(x_p, w1)


def _tc1_body(xw_ref, ds_ref, di_ref, h_ref, no_ref, ni_ref):
    no = _norms_from(ds_ref[0], ds_ref[1])
    ni = _norms_from(di_ref[0], di_ref[1])
    no_ref[...] = no
    ni_ref[...] = ni
    h_ref[...] = xw_ref[...] * no


def _tc1(xw, ds, di):
    vec3 = pl.BlockSpec((NC, ROWBLK, 1), lambda i: (0, i, 0))
    vec = pl.BlockSpec((ROWBLK, 1), lambda i: (i, 0))
    return pl.pallas_call(
        _tc1_body,
        grid=(NBLK,),
        in_specs=[
            pl.BlockSpec((ROWBLK, 128), lambda i: (i, 0)),
            vec3, vec3,
        ],
        out_specs=[pl.BlockSpec((ROWBLK, 128), lambda i: (i, 0)), vec, vec],
        out_shape=[
            jax.ShapeDtypeStruct((N_PAD, 128), jnp.float32),
            jax.ShapeDtypeStruct((N_PAD, 1), jnp.float32),
            jax.ShapeDtypeStruct((N_PAD, 1), jnp.float32),
        ],
    )(xw, ds, di)


def _tc2_body(p_ref, ni, no, b1r, w2, h2_ref):
    agg = p_ref[0] + p_ref[1]
    nic = ni[...]
    noc = no[...]
    t = jnp.maximum(agg * nic + b1r[...], 0.0)
    h2_ref[...] = jnp.dot(t * noc, w2[...], preferred_element_type=jnp.float32)


def _tc2(p, ni, no, b1r, w2):
    vec = pl.BlockSpec((ROWBLK, 1), lambda i: (i, 0))
    return pl.pallas_call(
        _tc2_body,
        grid=(NBLK,),
        in_specs=[
            pl.BlockSpec((NC, ROWBLK, 128), lambda i: (0, i, 0)),
            vec, vec,
            pl.BlockSpec((1, 128), lambda i: (0, 0)),
            pl.BlockSpec((128, 64), lambda i: (0, 0)),
        ],
        out_specs=pl.BlockSpec((ROWBLK, 64), lambda i: (i, 0)),
        out_shape=jax.ShapeDtypeStruct((N_PAD, 64), jnp.float32),
    )(p, ni, no, b1r, w2)


def _tc3_body(p_ref, ni, b2r, out_ref):
    z = p_ref[0] + p_ref[1]
    nic = ni[...]
    z = z * nic + b2r[...]
    z = jnp.maximum(z, 0.0)
    m = jnp.max(z, axis=1, keepdims=True)
    e = jnp.exp(z - m)
    s = jnp.sum(e, axis=1, keepdims=True)
    out_ref[...] = (z - m) - jnp.log(s)


def _tc3(p, ni, b2r):
    vec = pl.BlockSpec((ROWBLK, 1), lambda i: (i, 0))
    return pl.pallas_call(
        _tc3_body,
        grid=(NBLK,),
        in_specs=[
            pl.BlockSpec((NC, ROWBLK, 64), lambda i: (0, i, 0)),
            vec,
            pl.BlockSpec((1, 64), lambda i: (0, 0)),
        ],
        out_specs=pl.BlockSpec((ROWBLK, 64), lambda i: (i, 0)),
        out_shape=jax.ShapeDtypeStruct((N_PAD, 64), jnp.float32),
    )(p, ni, b2r)


def kernel(features, edge_index, W1, b1, W2, b2):
    n, d_in = features.shape
    e = edge_index.shape[1]
    src = edge_index[0].astype(jnp.int32)
    dst = edge_index[1].astype(jnp.int32)
    pad_e = E_PAD - e
    pad_fill = n + jnp.arange(pad_e, dtype=jnp.int32) % (N_PAD - n)
    src_p = jnp.concatenate([src, pad_fill])
    dst_p = jnp.concatenate([dst, pad_fill])
    packed = src_p | (dst_p << 14)
    pk_deg = packed.reshape(NW, EPT)
    pk_agg = packed.reshape(NW, NCHUNK, CHUNK)
    x_p = jnp.pad(features, ((0, N_PAD - n), (0, 0)))
    zeros128 = jnp.zeros((N_PAD, 128), jnp.float32)
    zeros64 = jnp.zeros((N_PAD, 64), jnp.float32)

    zd = jnp.zeros((DEGSZ,), jnp.float32)
    degp = _deg_kernel(pk_deg, zd)
    xw = _mm1(x_p, W1)
    degc = degp.reshape(NC, 2, N_PAD, 1)
    ds = degc[:, 0]
    di = degc[:, 1]

    h1, no, ni = _tc1(xw, ds, di)
    p1 = _agg_kernel(h1, pk_agg, zeros128)
    h2 = _tc2(p1, ni, no, b1.reshape(1, 128), W2)
    p2 = _agg_kernel64(h2, pk_agg, zeros64)
    out = _tc3(p2, ni, b2.reshape(1, 64))
    return out[:n]


# speedup vs baseline: 1.1032x; 1.0052x over previous
import functools

import jax
import jax.numpy as jnp
from jax import lax
from jax.experimental import pallas as pl
from jax.experimental.pallas import tpu as pltpu
from jax.experimental.pallas import tpu_sc as plsc

N_PAD = 10240
NC, NS = 2, 16
NW = NC * NS
CHUNK = 128
E_PAD = 327680
EPT = E_PAD // NW
NCHUNK = EPT // CHUNK
STRIPE = N_PAD // NS
ROWBLK = 1000
NBLK = 10
PMASK = 16383

_SC_PARAMS = pltpu.CompilerParams(needs_layout_passes=False)
_SC_MESH = dict(core_axis_name="c", subcore_axis_name="s")


DEGSZ = 2 * N_PAD
DSTRIPE = DEGSZ // NS


def _deg_body(pk_hbm, zd_hbm, out_hbm, idx_v, deg_v, acc_v, tmp_v, parts_sh):
    cid = lax.axis_index("c")
    sid = lax.axis_index("s")
    wid = cid * NS + sid
    pltpu.sync_copy(zd_hbm, deg_v)

    ones = jnp.full((16,), 1.0, jnp.float32)
    pltpu.sync_copy(pk_hbm.at[wid], idx_v)

    def acc(i, carry):
        v = idx_v[pl.ds(i * 16, 16)]
        plsc.addupdate_scatter(deg_v, [v & PMASK], ones)
        plsc.addupdate_scatter(deg_v, [(v >> 14) + N_PAD], ones)
        return carry

    lax.fori_loop(0, EPT // 16, acc, 0)

    pltpu.sync_copy(deg_v, parts_sh.at[sid])
    plsc.subcore_barrier()

    s0 = sid * DSTRIPE
    pltpu.sync_copy(parts_sh.at[:, pl.ds(s0, DSTRIPE)], tmp_v)

    def red(i, carry):
        sl = pl.ds(i * 16, 16)
        s = tmp_v[0, sl]
        for p in range(1, NS):
            s = s + tmp_v[p, sl]
        acc_v[sl] = s
        return carry

    lax.fori_loop(0, DSTRIPE // 16, red, 0)
    pltpu.sync_copy(acc_v, out_hbm.at[cid, pl.ds(s0, DSTRIPE)])


_deg_kernel = functools.partial(
    pl.kernel,
    out_type=jax.ShapeDtypeStruct((NC, DEGSZ), jnp.float32),
    mesh=plsc.VectorSubcoreMesh(**_SC_MESH),
    compiler_params=_SC_PARAMS,
    scratch_types=[
        pltpu.VMEM((EPT,), jnp.int32),
        pltpu.VMEM((DEGSZ,), jnp.float32),
        pltpu.VMEM((DSTRIPE,), jnp.float32),
        pltpu.VMEM((NS, DSTRIPE), jnp.float32),
        pltpu.VMEM_SHARED((NS, DEGSZ), jnp.float32),
    ],
)(_deg_body)


def _agg_body(h_hbm, pk_hbm, zero_hbm, out_hbm,
              pk_v, si0, si1, di0, di1, buf_v, agg_sh, sem0, sem1):
    cid = lax.axis_index("c")
    sid = lax.axis_index("s")
    wid = cid * NS + sid
    pltpu.sync_copy(pk_hbm.at[wid], pk_v)

    sidx = (si0, si1)
    didx = (di0, di1)

    def unpack(j, slot):
        for k in range(CHUNK // 16):
            sl = pl.ds(k * 16, 16)
            v = pk_v[j, sl]
            sidx[slot][sl] = v & PMASK
            didx[slot][sl] = v >> 14

    r0 = sid * STRIPE
    pltpu.sync_copy(zero_hbm.at[pl.ds(r0, STRIPE)], agg_sh.at[pl.ds(r0, STRIPE)])
    plsc.subcore_barrier()

    unpack(0, 0)
    pltpu.async_copy(h_hbm.at[si0], buf_v.at[0], sem0)

    def step(i, carry):
        j0 = 2 * i
        j1 = 2 * i + 1
        unpack(j1, 1)
        pltpu.async_copy(h_hbm.at[si1], buf_v.at[1], sem1)
        pltpu.make_async_copy(h_hbm.at[si0], buf_v.at[0], sem0).wait()
        pltpu.sync_copy(buf_v.at[0], agg_sh.at[di0], add=True)

        @pl.when(j1 + 1 < NCHUNK)
        def _():
            unpack(j1 + 1, 0)
            pltpu.async_copy(h_hbm.at[si0], buf_v.at[0], sem0)

        pltpu.make_async_copy(h_hbm.at[si1], buf_v.at[1], sem1).wait()
        pltpu.sync_copy(buf_v.at[1], agg_sh.at[di1], add=True)
        return carry

    lax.fori_loop(0, NCHUNK // 2, step, 0)
    plsc.subcore_barrier()
    pltpu.sync_copy(agg_sh.at[pl.ds(r0, STRIPE)],
                    out_hbm.at[cid, pl.ds(r0, STRIPE)])


def _make_agg_kernel(dh, tc_tiling):
    return functools.partial(
        pl.kernel,
        out_type=jax.ShapeDtypeStruct((NC, N_PAD, dh), jnp.float32),
        mesh=plsc.VectorSubcoreMesh(**_SC_MESH),
        compiler_params=pltpu.CompilerParams(
            needs_layout_passes=False, use_tc_tiling_on_sc=tc_tiling),
        scratch_types=[
            pltpu.VMEM((NCHUNK, CHUNK), jnp.int32),
            pltpu.VMEM((CHUNK,), jnp.int32),
            pltpu.VMEM((CHUNK,), jnp.int32),
            pltpu.VMEM((CHUNK,), jnp.int32),
            pltpu.VMEM((CHUNK,), jnp.int32),
            pltpu.VMEM((2, CHUNK, dh), jnp.float32),
            pltpu.VMEM_SHARED((N_PAD, dh), jnp.float32),
            pltpu.SemaphoreType.DMA,
            pltpu.SemaphoreType.DMA,
        ],
    )(_agg_body)


_agg_kernel = _make_agg_kernel(128, None)
_agg_kernel64 = _make_agg_kernel(64, False)


def _norms_from(deg_a, deg_b):
    deg = deg_a + deg_b
    return jnp.where(deg > 0, lax.rsqrt(jnp.maximum(deg, 1.0)), 0.0)


def _mm1_body(x_ref, w_ref, xw_ref):
    xw_ref[...] = jnp.dot(x_ref[...], w_ref[...],
                          preferred_element_type=jnp.float32)


def _mm1(x, w1):
    return pl.pallas_call(
        _mm1_body,
        grid=(NBLK,),
        in_specs=[
            pl.BlockSpec((ROWBLK, 128), lambda i: (i, 0)),
            pl.BlockSpec((128, 128), lambda i: (0, 0)),
        ],
        out_specs=pl.BlockSpec((ROWBLK, 128), lambda i: (i, 0)),
        out_shape=jax.ShapeDtypeStruct((N_PAD, 128), jnp.float32),
    )(x, w1)


def _tc1_body(xw_ref, ds_ref, di_ref, h_ref, no_ref, ni_ref):
    no = _norms_from(ds_ref[0], ds_ref[1])
    ni = _norms_from(di_ref[0], di_ref[1])
    no_ref[...] = no
    ni_ref[...] = ni
    h_ref[...] = xw_ref[...] * no


def _tc1(xw, ds, di):
    vec3 = pl.BlockSpec((NC, ROWBLK, 1), lambda i: (0, i, 0))
    vec = pl.BlockSpec((ROWBLK, 1), lambda i: (i, 0))
    return pl.pallas_call(
        _tc1_body,
        grid=(NBLK,),
        in_specs=[
            pl.BlockSpec((ROWBLK, 128), lambda i: (i, 0)),
            vec3, vec3,
        ],
        out_specs=[pl.BlockSpec((ROWBLK, 128), lambda i: (i, 0)), vec, vec],
        out_shape=[
            jax.ShapeDtypeStruct((N_PAD, 128), jnp.float32),
            jax.ShapeDtypeStruct((N_PAD, 1), jnp.float32),
            jax.ShapeDtypeStruct((N_PAD, 1), jnp.float32),
        ],
    )(xw, ds, di)


def _tc2_body(p_ref, ni, no, b1r, w2, h2_ref):
    agg = p_ref[0] + p_ref[1]
    nic = ni[...]
    noc = no[...]
    t = jnp.maximum(agg * nic + b1r[...], 0.0)
    h2_ref[...] = jnp.dot(t * noc, w2[...], preferred_element_type=jnp.float32)


def _tc2(p, ni, no, b1r, w2):
    vec = pl.BlockSpec((ROWBLK, 1), lambda i: (i, 0))
    return pl.pallas_call(
        _tc2_body,
        grid=(NBLK,),
        in_specs=[
            pl.BlockSpec((NC, ROWBLK, 128), lambda i: (0, i, 0)),
            vec, vec,
            pl.BlockSpec((1, 128), lambda i: (0, 0)),
            pl.BlockSpec((128, 64), lambda i: (0, 0)),
        ],
        out_specs=pl.BlockSpec((ROWBLK, 64), lambda i: (i, 0)),
        out_shape=jax.ShapeDtypeStruct((N_PAD, 64), jnp.float32),
    )(p, ni, no, b1r, w2)


def _tc3_body(p_ref, ni, b2r, out_ref):
    z = p_ref[0] + p_ref[1]
    nic = ni[...]
    z = z * nic + b2r[...]
    z = jnp.maximum(z, 0.0)
    m = jnp.max(z, axis=1, keepdims=True)
    e = jnp.exp(z - m)
    s = jnp.sum(e, axis=1, keepdims=True)
    out_ref[...] = (z - m) - jnp.log(s)


def _tc3(p, ni, b2r):
    vec = pl.BlockSpec((ROWBLK, 1), lambda i: (i, 0))
    return pl.pallas_call(
        _tc3_body,
        grid=(NBLK,),
        in_specs=[
            pl.BlockSpec((NC, ROWBLK, 64), lambda i: (0, i, 0)),
            vec,
            pl.BlockSpec((1, 64), lambda i: (0, 0)),
        ],
        out_specs=pl.BlockSpec((ROWBLK, 64), lambda i: (i, 0)),
        out_shape=jax.ShapeDtypeStruct((10 * ROWBLK, 64), jnp.float32),
    )(p, ni, b2r)


def kernel(features, edge_index, W1, b1, W2, b2):
    n, d_in = features.shape
    e = edge_index.shape[1]
    src = edge_index[0].astype(jnp.int32)
    dst = edge_index[1].astype(jnp.int32)
    pad_e = E_PAD - e
    pad_fill = n + jnp.arange(pad_e, dtype=jnp.int32) % (N_PAD - n)
    src_p = jnp.concatenate([src, pad_fill])
    dst_p = jnp.concatenate([dst, pad_fill])
    packed = src_p | (dst_p << 14)
    pk_deg = packed.reshape(NW, EPT)
    pk_agg = packed.reshape(NW, NCHUNK, CHUNK)
    zeros128 = jnp.zeros((N_PAD, 128), jnp.float32)
    zeros64 = jnp.zeros((N_PAD, 64), jnp.float32)

    zd = jnp.zeros((DEGSZ,), jnp.float32)
    degp = _deg_kernel(pk_deg, zd)
    xw = _mm1(features, W1)
    degc = degp.reshape(NC, 2, N_PAD, 1)
    ds = degc[:, 0]
    di = degc[:, 1]

    h1, no, ni = _tc1(xw, ds, di)
    p1 = _agg_kernel(h1, pk_agg, zeros128)
    h2 = _tc2(p1, ni, no, b1.reshape(1, 128), W2)
    p2 = _agg_kernel64(h2, pk_agg, zeros64)
    out = _tc3(p2, ni, b2.reshape(1, 64))
    return out


# speedup vs baseline: 1.1050x; 1.0016x over previous
import functools

import jax
import jax.numpy as jnp
from jax import lax
from jax.experimental import pallas as pl
from jax.experimental.pallas import tpu as pltpu
from jax.experimental.pallas import tpu_sc as plsc

N_PAD = 10240
NC, NS = 2, 16
NW = NC * NS
CHUNK = 128
E_PAD = 327680
EPT = E_PAD // NW
NCHUNK = EPT // CHUNK
STRIPE = N_PAD // NS
ROWBLK = 1000
NBLK = 10
PMASK = 16383

_SC_PARAMS = pltpu.CompilerParams(needs_layout_passes=False)
_SC_MESH = dict(core_axis_name="c", subcore_axis_name="s")


DEGSZ = 2 * N_PAD
DSTRIPE = DEGSZ // NS


def _deg_body(pk_hbm, zd_hbm, out_hbm, idx_v, deg_v, acc_v, tmp_v, parts_sh):
    cid = lax.axis_index("c")
    sid = lax.axis_index("s")
    wid = cid * NS + sid
    pltpu.sync_copy(zd_hbm, deg_v)

    ones = jnp.full((16,), 1.0, jnp.float32)
    pltpu.sync_copy(pk_hbm.at[wid], idx_v)

    def acc(i, carry):
        v = idx_v[pl.ds(i * 16, 16)]
        plsc.addupdate_scatter(deg_v, [v & PMASK], ones)
        plsc.addupdate_scatter(deg_v, [(v >> 14) + N_PAD], ones)
        return carry

    lax.fori_loop(0, EPT // 16, acc, 0)

    pltpu.sync_copy(deg_v, parts_sh.at[sid])
    plsc.subcore_barrier()

    s0 = sid * DSTRIPE
    pltpu.sync_copy(parts_sh.at[:, pl.ds(s0, DSTRIPE)], tmp_v)

    def red(i, carry):
        sl = pl.ds(i * 16, 16)
        s = tmp_v[0, sl]
        for p in range(1, NS):
            s = s + tmp_v[p, sl]
        acc_v[sl] = s
        return carry

    lax.fori_loop(0, DSTRIPE // 16, red, 0)
    pltpu.sync_copy(acc_v, out_hbm.at[cid, pl.ds(s0, DSTRIPE)])


_deg_kernel = functools.partial(
    pl.kernel,
    out_type=jax.ShapeDtypeStruct((NC, DEGSZ), jnp.float32),
    mesh=plsc.VectorSubcoreMesh(**_SC_MESH),
    compiler_params=_SC_PARAMS,
    scratch_types=[
        pltpu.VMEM((EPT,), jnp.int32),
        pltpu.VMEM((DEGSZ,), jnp.float32),
        pltpu.VMEM((DSTRIPE,), jnp.float32),
        pltpu.VMEM((NS, DSTRIPE), jnp.float32),
        pltpu.VMEM_SHARED((NS, DEGSZ), jnp.float32),
    ],
)(_deg_body)


def _agg_body(h_hbm, pk_hbm, zero_hbm, out_hbm,
              pk_v, si0, si1, di0, di1, buf_v, agg_sh, sem0, sem1):
    cid = lax.axis_index("c")
    sid = lax.axis_index("s")
    wid = cid * NS + sid
    pltpu.sync_copy(pk_hbm.at[wid], pk_v)

    sidx = (si0, si1)
    didx = (di0, di1)

    def unpack(j, slot):
        for k in range(CHUNK // 16):
            sl = pl.ds(k * 16, 16)
            v = pk_v[j, sl]
            sidx[slot][sl] = v & PMASK
            didx[slot][sl] = v >> 14

    r0 = sid * STRIPE
    pltpu.sync_copy(zero_hbm.at[pl.ds(r0, STRIPE)], agg_sh.at[pl.ds(r0, STRIPE)])
    plsc.subcore_barrier()

    unpack(0, 0)
    pltpu.async_copy(h_hbm.at[si0], buf_v.at[0], sem0)

    def step(i, carry):
        j0 = 2 * i
        j1 = 2 * i + 1
        unpack(j1, 1)
        pltpu.async_copy(h_hbm.at[si1], buf_v.at[1], sem1)
        pltpu.make_async_copy(h_hbm.at[si0], buf_v.at[0], sem0).wait()
        pltpu.sync_copy(buf_v.at[0], agg_sh.at[di0], add=True)

        @pl.when(j1 + 1 < NCHUNK)
        def _():
            unpack(j1 + 1, 0)
            pltpu.async_copy(h_hbm.at[si0], buf_v.at[0], sem0)

        pltpu.make_async_copy(h_hbm.at[si1], buf_v.at[1], sem1).wait()
        pltpu.sync_copy(buf_v.at[1], agg_sh.at[di1], add=True)
        return carry

    lax.fori_loop(0, NCHUNK // 2, step, 0)
    plsc.subcore_barrier()
    pltpu.sync_copy(agg_sh.at[pl.ds(r0, STRIPE)],
                    out_hbm.at[cid, pl.ds(r0, STRIPE)])


def _make_agg_kernel(dh, tc_tiling):
    return functools.partial(
        pl.kernel,
        out_type=jax.ShapeDtypeStruct((NC, N_PAD, dh), jnp.float32),
        mesh=plsc.VectorSubcoreMesh(**_SC_MESH),
        compiler_params=pltpu.CompilerParams(
            needs_layout_passes=False, use_tc_tiling_on_sc=tc_tiling),
        scratch_types=[
            pltpu.VMEM((NCHUNK, CHUNK), jnp.int32),
            pltpu.VMEM((CHUNK,), jnp.int32),
            pltpu.VMEM((CHUNK,), jnp.int32),
            pltpu.VMEM((CHUNK,), jnp.int32),
            pltpu.VMEM((CHUNK,), jnp.int32),
            pltpu.VMEM((2, CHUNK, dh), jnp.float32),
            pltpu.VMEM_SHARED((N_PAD, dh), jnp.float32),
            pltpu.SemaphoreType.DMA,
            pltpu.SemaphoreType.DMA,
        ],
    )(_agg_body)


_agg_kernel = _make_agg_kernel(128, False)
_agg_kernel64 = _make_agg_kernel(64, False)


def _norms_from(deg_a, deg_b):
    deg = deg_a + deg_b
    return jnp.where(deg > 0, lax.rsqrt(jnp.maximum(deg, 1.0)), 0.0)


def _mm1_body(x_ref, w_ref, xw_ref):
    xw_ref[...] = jnp.dot(x_ref[...], w_ref[...],
                          preferred_element_type=jnp.float32)


def _mm1(x, w1):
    return pl.pallas_call(
        _mm1_body,
        grid=(NBLK,),
        in_specs=[
            pl.BlockSpec((ROWBLK, 128), lambda i: (i, 0)),
            pl.BlockSpec((128, 128), lambda i: (0, 0)),
        ],
        out_specs=pl.BlockSpec((ROWBLK, 128), lambda i: (i, 0)),
        out_shape=jax.ShapeDtypeStruct((N_PAD, 128), jnp.float32),
    )(x, w1)


def _tc1_body(xw_ref, ds_ref, di_ref, h_ref, no_ref, ni_ref):
    no = _norms_from(ds_ref[0], ds_ref[1])
    ni = _norms_from(di_ref[0], di_ref[1])
    no_ref[...] = no
    ni_ref[...] = ni
    h_ref[...] = xw_ref[...] * no


def _tc1(xw, ds, di):
    vec3 = pl.BlockSpec((NC, ROWBLK, 1), lambda i: (0, i, 0))
    vec = pl.BlockSpec((ROWBLK, 1), lambda i: (i, 0))
    return pl.pallas_call(
        _tc1_body,
        grid=(NBLK,),
        in_specs=[
            pl.BlockSpec((ROWBLK, 128), lambda i: (i, 0)),
            vec3, vec3,
        ],
        out_specs=[pl.BlockSpec((ROWBLK, 128), lambda i: (i, 0)), vec, vec],
        out_shape=[
            jax.ShapeDtypeStruct((N_PAD, 128), jnp.float32),
            jax.ShapeDtypeStruct((N_PAD, 1), jnp.float32),
            jax.ShapeDtypeStruct((N_PAD, 1), jnp.float32),
        ],
    )(xw, ds, di)


def _tc2_body(p_ref, ni, no, b1r, w2, h2_ref):
    agg = p_ref[0] + p_ref[1]
    nic = ni[...]
    noc = no[...]
    t = jnp.maximum(agg * nic + b1r[...], 0.0)
    h2_ref[...] = jnp.dot(t * noc, w2[...], preferred_element_type=jnp.float32)


def _tc2(p, ni, no, b1r, w2):
    vec = pl.BlockSpec((ROWBLK, 1), lambda i: (i, 0))
    return pl.pallas_call(
        _tc2_body,
        grid=(NBLK,),
        in_specs=[
            pl.BlockSpec((NC, ROWBLK, 128), lambda i: (0, i, 0)),
            vec, vec,
            pl.BlockSpec((1, 128), lambda i: (0, 0)),
            pl.BlockSpec((128, 64), lambda i: (0, 0)),
        ],
        out_specs=pl.BlockSpec((ROWBLK, 64), lambda i: (i, 0)),
        out_shape=jax.ShapeDtypeStruct((N_PAD, 64), jnp.float32),
    )(p, ni, no, b1r, w2)


def _tc3_body(p_ref, ni, b2r, out_ref):
    z = p_ref[0] + p_ref[1]
    nic = ni[...]
    z = z * nic + b2r[...]
    z = jnp.maximum(z, 0.0)
    m = jnp.max(z, axis=1, keepdims=True)
    e = jnp.exp(z - m)
    s = jnp.sum(e, axis=1, keepdims=True)
    out_ref[...] = (z - m) - jnp.log(s)


def _tc3(p, ni, b2r):
    vec = pl.BlockSpec((ROWBLK, 1), lambda i: (i, 0))
    return pl.pallas_call(
        _tc3_body,
        grid=(NBLK,),
        in_specs=[
            pl.BlockSpec((NC, ROWBLK, 64), lambda i: (0, i, 0)),
            vec,
            pl.BlockSpec((1, 64), lambda i: (0, 0)),
        ],
        out_specs=pl.BlockSpec((ROWBLK, 64), lambda i: (i, 0)),
        out_shape=jax.ShapeDtypeStruct((10 * ROWBLK, 64), jnp.float32),
    )(p, ni, b2r)


def kernel(features, edge_index, W1, b1, W2, b2):
    n, d_in = features.shape
    e = edge_index.shape[1]
    src = edge_index[0].astype(jnp.int32)
    dst = edge_index[1].astype(jnp.int32)
    pad_e = E_PAD - e
    pad_fill = n + jnp.arange(pad_e, dtype=jnp.int32) % (N_PAD - n)
    src_p = jnp.concatenate([src, pad_fill])
    dst_p = jnp.concatenate([dst, pad_fill])
    packed = src_p | (dst_p << 14)
    pk_deg = packed.reshape(NW, EPT)
    pk_agg = packed.reshape(NW, NCHUNK, CHUNK)
    zeros128 = jnp.zeros((N_PAD, 128), jnp.float32)
    zeros64 = jnp.zeros((N_PAD, 64), jnp.float32)

    zd = jnp.zeros((DEGSZ,), jnp.float32)
    degp = _deg_kernel(pk_deg, zd)
    xw = _mm1(features, W1)
    degc = degp.reshape(NC, 2, N_PAD, 1)
    ds = degc[:, 0]
    di = degc[:, 1]

    h1, no, ni = _tc1(xw, ds, di)
    p1 = _agg_kernel(h1, pk_agg, zeros128)
    h2 = _tc2(p1, ni, no, b1.reshape(1, 128), W2)
    p2 = _agg_kernel64(h2, pk_agg, zeros64)
    out = _tc3(p2, ni, b2.reshape(1, 64))
    return out
